# baseline (speedup 1.0000x reference)
import functools

import jax
import jax.numpy as jnp
from jax import lax
from jax.experimental import pallas as pl
from jax.experimental.pallas import tpu as pltpu
from jax.experimental.pallas import tpu_sc as plsc

NUM_NODES = 100000
EMBED_DIM = 128
BATCH = 16384

_info = plsc.get_sparse_core_info()
_NC = _info.num_cores
_NS = _info.num_subcores
_NW = _NC * _NS
_B_PER_W = BATCH // _NW

_NCHUNK = 8
_CH = _B_PER_W // _NCHUNK

_mesh = plsc.VectorSubcoreMesh(core_axis_name="c", subcore_axis_name="s")


@functools.partial(
    pl.kernel,
    mesh=_mesh,
    out_type=jax.ShapeDtypeStruct((BATCH, EMBED_DIM), jnp.float32),
    scratch_types=[
        pltpu.VMEM((_B_PER_W,), jnp.int32),
        pltpu.VMEM((_B_PER_W, EMBED_DIM), jnp.float32),
    ]
    + [pltpu.SemaphoreType.DMA] * (2 * _NCHUNK),
)
def _gather_kernel(table_hbm, idx_hbm, out_hbm, idx_v, rows_v, *sems):
    gsems = sems[:_NCHUNK]
    ssems = sems[_NCHUNK:]
    wid = lax.axis_index("s") * _NC + lax.axis_index("c")
    base = wid * _B_PER_W

    pltpu.sync_copy(idx_hbm.at[pl.ds(base, _CH)], idx_v.at[pl.ds(0, _CH)])
    gathers = [None] * _NCHUNK

    def issue_gather(i):
        return pltpu.async_copy(
            table_hbm.at[idx_v.at[pl.ds(i * _CH, _CH)]],
            rows_v.at[pl.ds(i * _CH, _CH)],
            gsems[i],
        )

    gathers[0] = issue_gather(0)
    if _NCHUNK > 1:
        pltpu.sync_copy(
            idx_hbm.at[pl.ds(base + _CH, _B_PER_W - _CH)],
            idx_v.at[pl.ds(_CH, _B_PER_W - _CH)],
        )
        for i in range(1, _NCHUNK):
            gathers[i] = issue_gather(i)

    stores = []
    for i in range(_NCHUNK):
        gathers[i].wait()
        stores.append(
            pltpu.async_copy(
                rows_v.at[pl.ds(i * _CH, _CH)],
                out_hbm.at[pl.ds(base + i * _CH, _CH)],
                ssems[i],
            )
        )
    for s in stores:
        s.wait()


def kernel(batch, embedding_weight):
    return _gather_kernel(embedding_weight, batch.astype(jnp.int32))


# speedup vs baseline: 1.0478x; 1.0478x over previous
import functools

import jax
import jax.numpy as jnp
from jax import lax
from jax.experimental import pallas as pl
from jax.experimental.pallas import tpu as pltpu
from jax.experimental.pallas import tpu_sc as plsc

NUM_NODES = 100000
EMBED_DIM = 128
BATCH = 16384

_info = plsc.get_sparse_core_info()
_NC = _info.num_cores
_NS = _info.num_subcores
_NW = _NC * _NS
_B_PER_W = BATCH // _NW

_mesh = plsc.VectorSubcoreMesh(core_axis_name="c", subcore_axis_name="s")


@functools.partial(
    pl.kernel,
    mesh=_mesh,
    out_type=jax.ShapeDtypeStruct((BATCH, EMBED_DIM), jnp.float32),
    scratch_types=[
        pltpu.VMEM((_B_PER_W,), jnp.int32),
        pltpu.VMEM((_B_PER_W, EMBED_DIM), jnp.float32),
        pltpu.SemaphoreType.DMA,
    ],
)
def _gather_kernel(table_hbm, idx_hbm, out_hbm, idx_v, rows_v, sem):
    wid = lax.axis_index("s") * _NC + lax.axis_index("c")
    base = wid * _B_PER_W
    pltpu.sync_copy(idx_hbm.at[pl.ds(base, _B_PER_W)], idx_v)
    pltpu.async_copy(table_hbm.at[idx_v], rows_v, sem).wait()
    pltpu.sync_copy(rows_v, out_hbm.at[pl.ds(base, _B_PER_W)])


def kernel(batch, embedding_weight):
    return _gather_kernel(embedding_weight, batch.astype(jnp.int32))
